# Initial kernel scaffold; baseline (speedup 1.0000x reference)
import jax
import jax.numpy as jnp
from jax.experimental import pallas as pl


def kernel(x, edge_index, edge_attr, We, be, lin0_W, lin0_b, m0_W1, m0_b1, m0_W2, m0_b2, bn0_g, bn0_b, lin1_W, lin1_b, m1_W1, m1_b1, m1_W2, m1_b2, bn1_g, bn1_b, lin2_W, lin2_b, m2_W1, m2_b1, m2_W2, m2_b2, bn2_g, bn2_b, pW1, pb1, pW2, pb2):
    raise NotImplementedError("write your pallas kernel here")


# speedup vs baseline: 2.6616x; 2.6616x over previous
import functools

import jax
import jax.numpy as jnp
from jax import lax
from jax.experimental import pallas as pl
from jax.experimental.pallas import tpu as pltpu
from jax.experimental.pallas import tpu_sc as plsc

_N = 10000
_E = 320000
_NW = 32
_EPW = _E // _NW
_C = 128
_EPWP = 10240
_NCH = _EPWP // _C
_EP = _NW * _EPWP
_D = 64
_NSL = _D // 16
_RWIN = 640
_RSTR = 624
_BN_EPS = 1e-5
_PAD = -1e38


def _make_sc_aggr():
    mesh = plsc.VectorSubcoreMesh(core_axis_name="c", subcore_axis_name="s")

    @functools.partial(
        pl.kernel,
        out_type=jax.ShapeDtypeStruct((2 * _N, _D), jnp.float32),
        mesh=mesh,
        compiler_params=pltpu.CompilerParams(use_tc_tiling_on_sc=False),
        scratch_types=[
            pltpu.VMEM_SHARED((_N, _D), jnp.float32),
            pltpu.VMEM((_NCH, _C), jnp.int32),
            pltpu.VMEM((_NCH, _C), jnp.int32),
            pltpu.VMEM((_C, _D), jnp.float32),
            pltpu.VMEM((_C, _D), jnp.float32),
            pltpu.VMEM((_C, _D), jnp.float32),
            pltpu.VMEM((_C, _D), jnp.float32),
            pltpu.SemaphoreType.DMA,
            pltpu.SemaphoreType.DMA,
            pltpu.SemaphoreType.DMA,
            pltpu.SemaphoreType.DMA,
        ],
    )
    def sc_aggr(h_hbm, ep_hbm, src_hbm, dst_hbm, out_hbm,
                accum, src_v, dst_v, rows0, rows1, ep0, ep1,
                g0, g1, e0, e1):
        c = lax.axis_index("c")
        s = lax.axis_index("s")
        wid = s * 2 + c
        rows = (rows0, rows1)
        eps = (ep0, ep1)
        gsem = (g0, g1)
        esem = (e0, e1)

        pltpu.sync_copy(src_hbm.at[wid], src_v)
        pltpu.sync_copy(dst_hbm.at[wid], dst_v)

        def zrow(r, carry):
            for j in range(_NSL):
                rows0[r, pl.ds(j * 16, 16)] = jnp.zeros((16,), jnp.float32)
            return carry
        lax.fori_loop(0, _C, zrow, 0)
        for t in range(_RWIN // _C):
            pltpu.sync_copy(rows0, accum.at[pl.ds(s * _RSTR + t * _C, _C)])
        plsc.subcore_barrier()

        ep_base = wid * _EPWP

        def start(i, k):
            pltpu.make_async_copy(h_hbm.at[src_v.at[i]], rows[k], gsem[k]).start()
            pltpu.make_async_copy(
                ep_hbm.at[pl.ds(ep_base + i * _C, _C)], eps[k], esem[k]).start()

        def step(i, k):
            pltpu.make_async_copy(h_hbm.at[src_v.at[i]], rows[k], gsem[k]).wait()
            pltpu.make_async_copy(
                ep_hbm.at[pl.ds(ep_base + i * _C, _C)], eps[k], esem[k]).wait()

            def crow(r, cc):
                for j in range(_NSL):
                    sl = (r, pl.ds(j * 16, 16))
                    rows[k][sl] = jnp.maximum(rows[k][sl] + eps[k][sl], 0.0)
                return cc
            lax.fori_loop(0, _C, crow, 0)

            pltpu.sync_copy(rows[k], accum.at[dst_v.at[i]], add=True)

        start(0, 0)

        def outer(i2, carry):
            i0 = i2 * 2
            for k in (0, 1):
                i = i0 + k
                nxt = i + 1

                @pl.when(nxt < _NCH)
                def _():
                    start(nxt, 1 - k)

                step(i, k)
            return carry

        lax.fori_loop(0, _NCH // 2, outer, 0)
        plsc.subcore_barrier()

        pltpu.sync_copy(accum.at[pl.ds(s * _RSTR, _RWIN)],
                        out_hbm.at[pl.ds(c * _N + s * _RSTR, _RWIN)])

    return sc_aggr


_SC_AGGR = _make_sc_aggr()


def _eproj(edge_attr, We, be, lin0_W, lin0_b, lin1_W, lin1_b, lin2_W, lin2_b):
    f32 = jnp.float32
    npad = _EPWP - _EPW

    def body(ea, We_, be_, W0a, b0a, W0b, b0b, W1, b1, W2, b2,
             o0a, o0b, o1, o2):
        t = jnp.dot(ea[...], We_[...], preferred_element_type=f32) + be_[...]
        pad = jnp.full((npad, _D), _PAD, f32)

        def proj(W, b):
            v = jnp.dot(t, W[...], preferred_element_type=f32) + b[...]
            return jnp.concatenate([v, pad], axis=0)

        o0a[...] = proj(W0a, b0a)
        o0b[...] = proj(W0b, b0b)
        o1[...] = proj(W1, b1)
        o2[...] = proj(W2, b2)

    full = lambda shape: pl.BlockSpec(shape, lambda i: (0,) * len(shape))
    out_spec = pl.BlockSpec((_EPWP, _D), lambda i: (i, 0))
    outs = _eproj_call = pl.pallas_call(
        body,
        grid=(_NW,),
        in_specs=[
            pl.BlockSpec((_EPW, 16), lambda i: (i, 0)),
            full((16, 64)), full((1, 64)),
            full((64, _D)), full((1, _D)),
            full((64, _D)), full((1, _D)),
            full((64, _D)), full((1, _D)),
            full((64, _D)), full((1, _D)),
        ],
        out_specs=[out_spec] * 4,
        out_shape=[jax.ShapeDtypeStruct((_EP, _D), f32)] * 4,
    )(edge_attr, We, be.reshape(1, -1),
      lin0_W[:, :64], lin0_b[:64].reshape(1, -1),
      lin0_W[:, 64:], lin0_b[64:].reshape(1, -1),
      lin1_W, lin1_b.reshape(1, -1),
      lin2_W, lin2_b.reshape(1, -1))
    return outs


def _bn_relu(z, g_, beta_):
    mu = jnp.mean(z, axis=0, keepdims=True)
    d = z - mu
    var = jnp.mean(d * d, axis=0, keepdims=True)
    return jnp.maximum(d * (g_ * lax.rsqrt(var + _BN_EPS)) + beta_, 0.0)


def _mlp_bn0(h, parts_a, parts_b, W1, b1, W2, b2, g, beta):
    f32 = jnp.float32
    N = _N

    def body(h_, a_, b_, W1_, b1_, W2_, b2_, g_, beta_, o_):
        aggr = jnp.concatenate(
            [a_[:N, :] + a_[N:, :], b_[:N, :] + b_[N:, :]], axis=1)
        out = h_[...] + aggr
        z = jnp.maximum(jnp.dot(out, W1_[...], preferred_element_type=f32)
                        + b1_[...], 0.0)
        z = jnp.dot(z, W2_[...], preferred_element_type=f32) + b2_[...]
        o_[...] = _bn_relu(z, g_[...], beta_[...])

    return pl.pallas_call(
        body,
        out_shape=jax.ShapeDtypeStruct((N, 64), f32),
    )(h, parts_a, parts_b, W1, b1.reshape(1, -1), W2, b2.reshape(1, -1),
      g.reshape(1, -1), beta.reshape(1, -1))


def _mlp_bn(h, parts, W1, b1, W2, b2, g, beta):
    f32 = jnp.float32
    N = _N

    def body(h_, a_, W1_, b1_, W2_, b2_, g_, beta_, o_):
        out = h_[...] + a_[:N, :] + a_[N:, :]
        z = jnp.maximum(jnp.dot(out, W1_[...], preferred_element_type=f32)
                        + b1_[...], 0.0)
        z = jnp.dot(z, W2_[...], preferred_element_type=f32) + b2_[...]
        o_[...] = _bn_relu(z, g_[...], beta_[...])

    return pl.pallas_call(
        body,
        out_shape=jax.ShapeDtypeStruct((N, 64), f32),
    )(h, parts, W1, b1.reshape(1, -1), W2, b2.reshape(1, -1),
      g.reshape(1, -1), beta.reshape(1, -1))


def _mlp_bn_final(h, parts, W1, b1, W2, b2, g, beta, pW1, pb1, pW2, pb2):
    f32 = jnp.float32
    N = _N

    def body(h_, a_, W1_, b1_, W2_, b2_, g_, beta_,
             pW1_, pb1_, pW2_, pb2_, o_):
        out = h_[...] + a_[:N, :] + a_[N:, :]
        z = jnp.maximum(jnp.dot(out, W1_[...], preferred_element_type=f32)
                        + b1_[...], 0.0)
        z = jnp.dot(z, W2_[...], preferred_element_type=f32) + b2_[...]
        hn = _bn_relu(z, g_[...], beta_[...])
        hm = jnp.mean(hn, axis=0, keepdims=True)
        p = jnp.maximum(jnp.dot(hm, pW1_[...], preferred_element_type=f32)
                        + pb1_[...], 0.0)
        o_[...] = jnp.dot(p, pW2_[...], preferred_element_type=f32) + pb2_[...]

    return pl.pallas_call(
        body,
        out_shape=jax.ShapeDtypeStruct((1, 64), f32),
    )(h, parts, W1, b1.reshape(1, -1), W2, b2.reshape(1, -1),
      g.reshape(1, -1), beta.reshape(1, -1),
      pW1, pb1.reshape(1, -1), pW2, pb2.reshape(1, -1))


def kernel(x, edge_index, edge_attr, We, be,
           lin0_W, lin0_b, m0_W1, m0_b1, m0_W2, m0_b2, bn0_g, bn0_b,
           lin1_W, lin1_b, m1_W1, m1_b1, m1_W2, m1_b2, bn1_g, bn1_b,
           lin2_W, lin2_b, m2_W1, m2_b1, m2_W2, m2_b2, bn2_g, bn2_b,
           pW1, pb1, pW2, pb2):
    npad = _EPWP - _EPW

    def pad_idx(row):
        r = row.reshape(_NW, _EPW)
        r = jnp.pad(r, ((0, 0), (0, npad)))
        return r.reshape(_NW, _NCH, _C)

    src = pad_idx(edge_index[0])
    dst = pad_idx(edge_index[1])

    ep0a, ep0b, ep1, ep2 = _eproj(edge_attr, We, be,
                                  lin0_W, lin0_b, lin1_W, lin1_b,
                                  lin2_W, lin2_b)

    h = x
    pa = _SC_AGGR(h[:, :64], ep0a, src, dst)
    pb = _SC_AGGR(h[:, 64:], ep0b, src, dst)
    h = _mlp_bn0(h, pa, pb, m0_W1, m0_b1, m0_W2, m0_b2, bn0_g, bn0_b)
    parts = _SC_AGGR(h, ep1, src, dst)
    h = _mlp_bn(h, parts, m1_W1, m1_b1, m1_W2, m1_b2, bn1_g, bn1_b)
    parts = _SC_AGGR(h, ep2, src, dst)
    return _mlp_bn_final(h, parts, m2_W1, m2_b1, m2_W2, m2_b2, bn2_g, bn2_b,
                         pW1, pb1, pW2, pb2)
